# baseline (speedup 1.0000x reference)
import functools

import jax
import jax.numpy as jnp
from jax import lax
from jax.experimental import pallas as pl
from jax.experimental.pallas import tpu as pltpu
from jax.experimental.pallas import tpu_sc as plsc

K = 3
BLKQ = 512
BLKR = 1024
NW = 32
CQ = 32
NBUF = 2


def _topk_body(n1_total, bs, p2_ref, p1_ref, idx_ref, w_ref):
    b = pl.program_id(0) + bs
    p2b = p2_ref[0]
    p1b = p1_ref[0]
    cross = lax.dot_general(p2b, p1b, (((0,), (0,)), ((), ())),
                            preferred_element_type=jnp.float32)
    n2sq = jnp.sum(p2b * p2b, axis=0)[:, None]
    n1sq = jnp.sum(p1b * p1b, axis=0)[None, :]
    d = jnp.maximum(n2sq + n1sq - 2.0 * cross, 0.0)
    iota = lax.broadcasted_iota(jnp.int32, d.shape, 1)
    idxs, gds = [], []
    for _ in range(K):
        m = jnp.min(d, axis=1, keepdims=True)
        ik = jnp.min(jnp.where(d == m, iota, n1_total), axis=1, keepdims=True)
        idxs.append(ik)
        gds.append(jnp.sqrt(m))
        d = jnp.where(iota == ik, jnp.inf, d)
    inv = [1.0 / (g + 1e-10) for g in gds]
    tot = inv[0] + inv[1] + inv[2]
    idx_ref[0] = jnp.concatenate(idxs, axis=1) + b * n1_total
    w_ref[0] = jnp.concatenate([v / tot for v in inv], axis=1)


def _topk(points2, points1, bs, nb):
    _, _, N2 = points2.shape
    N1 = points1.shape[2]
    grid = (nb, N2 // BLKQ)
    return pl.pallas_call(
        functools.partial(_topk_body, N1, bs),
        grid=grid,
        in_specs=[
            pl.BlockSpec((1, 3, BLKQ), lambda b, j: (b + bs, 0, j)),
            pl.BlockSpec((1, 3, N1), lambda b, j: (b + bs, 0, 0)),
        ],
        out_specs=[
            pl.BlockSpec((1, BLKQ, K), lambda b, j: (b, j, 0)),
            pl.BlockSpec((1, BLKQ, K), lambda b, j: (b, j, 0)),
        ],
        out_shape=[
            jax.ShapeDtypeStruct((nb, N2, K), jnp.int32),
            jax.ShapeDtypeStruct((nb, N2, K), jnp.float32),
        ],
        compiler_params=pltpu.CompilerParams(
            dimension_semantics=("parallel", "parallel")),
    )(points2, points1)


def _proj_body(f1_ref, wt_ref, o_ref):
    o_ref[...] = lax.dot_general(f1_ref[0], wt_ref[...],
                                 (((0,), (0,)), ((), ())),
                                 preferred_element_type=jnp.float32)


def _proj(features1, wt, blk):
    B, C1, N1 = features1.shape
    Cout = wt.shape[1]
    jb = N1 // blk
    return pl.pallas_call(
        _proj_body,
        grid=(B, jb),
        in_specs=[
            pl.BlockSpec((1, C1, blk), lambda b, j: (b, 0, j)),
            pl.BlockSpec((C1, Cout), lambda b, j: (0, 0)),
        ],
        out_specs=pl.BlockSpec((blk, Cout), lambda b, j: (b * jb + j, 0)),
        out_shape=jax.ShapeDtypeStruct((B * N1, Cout), jnp.float32),
        compiler_params=pltpu.CompilerParams(
            dimension_semantics=("parallel", "parallel")),
    )(features1, wt)


def _interp_sc(table, gidx, wexp):
    V, C = table.shape
    QK = gidx.shape[0]
    Q = QK // K
    qw = Q // NW
    nch = qw // CQ
    rows = CQ * K
    cvec = C // 16

    mesh = plsc.VectorSubcoreMesh(core_axis_name="c", subcore_axis_name="s")

    @functools.partial(
        pl.kernel,
        out_type=jax.ShapeDtypeStruct((Q, C), jnp.float32),
        mesh=mesh,
        scratch_types=[
            pltpu.VMEM((qw * K,), jnp.int32),
            [pltpu.VMEM((rows, 16), jnp.float32) for _ in range(NBUF)],
            [pltpu.VMEM((rows, C), jnp.float32) for _ in range(NBUF)],
            [pltpu.SemaphoreType.DMA for _ in range(3 * NBUF)],
        ],
    )
    def k(table_hbm, gidx_hbm, wexp_hbm, out_hbm, idx_all, wbufs, rbufs,
          sems):
        wid = lax.axis_index("s") * 2 + lax.axis_index("c")
        base_q = wid * qw
        gsems = sems[0:NBUF]
        wsems = sems[NBUF:2 * NBUF]
        osems = sems[2 * NBUF:3 * NBUF]
        pltpu.sync_copy(gidx_hbm.at[pl.ds(base_q * K, qw * K)], idx_all)

        def start(ci, sl):
            r0 = ci * rows
            pltpu.async_copy(table_hbm.at[idx_all.at[pl.ds(r0, rows)]],
                             rbufs[sl], gsems[sl])
            pltpu.async_copy(wexp_hbm.at[pl.ds(base_q * K + r0, rows)],
                             wbufs[sl], wsems[sl])

        def wait_in(ci, sl):
            r0 = ci * rows
            pltpu.make_async_copy(table_hbm.at[idx_all.at[pl.ds(r0, rows)]],
                                  rbufs[sl], gsems[sl]).wait()
            pltpu.make_async_copy(wexp_hbm.at[pl.ds(base_q * K + r0, rows)],
                                  wbufs[sl], wsems[sl]).wait()

        def compute(ci, sl):
            rv, wv = rbufs[sl], wbufs[sl]

            def per_q(q, c2):
                rq = q * K
                wa = wv[rq, :]
                wb = wv[rq + 1, :]
                wc = wv[rq + 2, :]
                for c in range(cvec):
                    s = pl.ds(16 * c, 16)
                    rv[q, s] = (wa * rv[rq, s] + wb * rv[rq + 1, s]
                                + wc * rv[rq + 2, s])
                return c2

            lax.fori_loop(0, CQ, per_q, 0)
            pltpu.async_copy(rbufs[sl].at[pl.ds(0, CQ)],
                             out_hbm.at[pl.ds(base_q + ci * CQ, CQ)],
                             osems[sl])

        def wait_out(sl):
            pltpu.make_async_copy(rbufs[sl].at[pl.ds(0, CQ)],
                                  out_hbm.at[pl.ds(base_q, CQ)],
                                  osems[sl]).wait()

        for sl in range(NBUF):
            start(sl, sl)

        def group(ig, c2):
            for sl in range(NBUF):
                ci = NBUF * ig + sl
                wait_in(ci, sl)
                compute(ci, sl)

                @pl.when(ci + NBUF < nch)
                def _():
                    wait_out(sl)
                    start(ci + NBUF, sl)

            return c2

        lax.fori_loop(0, nch // NBUF, group, 0)
        for sl in range(NBUF):
            wait_out(sl)

    return k(table, gidx, wexp)


def _accum_stats(st_ref, y):
    s = jnp.sum(y, axis=0, keepdims=True)
    s2 = jnp.sum(y * y, axis=0, keepdims=True)
    st = jnp.concatenate([s, s2], axis=0)

    @pl.when(pl.program_id(0) == 0)
    def _():
        st_ref[...] = st

    @pl.when(pl.program_id(0) != 0)
    def _():
        st_ref[...] += st


def _l1_body(jb, ya_ref, f2_ref, sk_ref, w2t_ref, w3t_ref, b_ref, o_ref,
             st_ref):
    dn = (((0,), (0,)), ((), ()))
    y = (ya_ref[...]
         + lax.dot_general(f2_ref[0], w2t_ref[...], dn,
                           preferred_element_type=jnp.float32)
         + lax.dot_general(sk_ref[0], w3t_ref[...], dn,
                           preferred_element_type=jnp.float32)
         + b_ref[...])
    o_ref[...] = y
    _accum_stats(st_ref, y)


def _layer1(ya, features2, skip_features, w2t, w3t, b, bs):
    R, C = ya.shape
    B, C2, N2 = features2.shape
    CS = skip_features.shape[1]
    jb = N2 // BLKR
    return pl.pallas_call(
        functools.partial(_l1_body, jb),
        grid=(R // BLKR,),
        in_specs=[
            pl.BlockSpec((BLKR, C), lambda i: (i, 0)),
            pl.BlockSpec((1, C2, BLKR), lambda i: (i // jb + bs, 0, i % jb)),
            pl.BlockSpec((1, CS, BLKR), lambda i: (i // jb + bs, 0, i % jb)),
            pl.BlockSpec((C2, C), lambda i: (0, 0)),
            pl.BlockSpec((CS, C), lambda i: (0, 0)),
            pl.BlockSpec((1, C), lambda i: (0, 0)),
        ],
        out_specs=[
            pl.BlockSpec((BLKR, C), lambda i: (i, 0)),
            pl.BlockSpec((2, C), lambda i: (0, 0)),
        ],
        out_shape=[
            jax.ShapeDtypeStruct((R, C), jnp.float32),
            jax.ShapeDtypeStruct((2, C), jnp.float32),
        ],
        compiler_params=pltpu.CompilerParams(
            dimension_semantics=("arbitrary",)),
    )(ya, features2, skip_features, w2t, w3t, b)


def _bn_relu(y_ref, stin_ref, gam_ref, bet_ref, ntot):
    mean = stin_ref[0:1, :] * (1.0 / ntot)
    ex2 = stin_ref[1:2, :] * (1.0 / ntot)
    s = gam_ref[...] * lax.rsqrt(ex2 - mean * mean + 1e-3)
    t = bet_ref[...] - mean * s
    return jnp.maximum(y_ref[...] * s + t, 0.0)


def _layer_body(ntot, y_ref, stin_ref, gam_ref, bet_ref, wt_ref, b_ref,
                o_ref, st_ref):
    x = _bn_relu(y_ref, stin_ref, gam_ref, bet_ref, ntot)
    y = jnp.dot(x, wt_ref[...], preferred_element_type=jnp.float32) + b_ref[...]
    o_ref[...] = y
    _accum_stats(st_ref, y)


def _layer(yprev, stin, gam, bet, wt, b, ntot):
    R, Cin = yprev.shape
    Cout = wt.shape[1]
    return pl.pallas_call(
        functools.partial(_layer_body, ntot),
        grid=(R // BLKR,),
        in_specs=[
            pl.BlockSpec((BLKR, Cin), lambda i: (i, 0)),
            pl.BlockSpec((2, Cin), lambda i: (0, 0)),
            pl.BlockSpec((1, Cin), lambda i: (0, 0)),
            pl.BlockSpec((1, Cin), lambda i: (0, 0)),
            pl.BlockSpec((Cin, Cout), lambda i: (0, 0)),
            pl.BlockSpec((1, Cout), lambda i: (0, 0)),
        ],
        out_specs=[
            pl.BlockSpec((BLKR, Cout), lambda i: (i, 0)),
            pl.BlockSpec((2, Cout), lambda i: (0, 0)),
        ],
        out_shape=[
            jax.ShapeDtypeStruct((R, Cout), jnp.float32),
            jax.ShapeDtypeStruct((2, Cout), jnp.float32),
        ],
        compiler_params=pltpu.CompilerParams(
            dimension_semantics=("arbitrary",)),
    )(yprev, stin, gam, bet, wt, b)


def _layer_t_body(ntot, y_ref, stin_ref, gam_ref, bet_ref, w_ref, b_ref,
                  o_ref, st_ref):
    x = _bn_relu(y_ref, stin_ref, gam_ref, bet_ref, ntot)
    yt = lax.dot_general(w_ref[...], x, (((1,), (1,)), ((), ())),
                         preferred_element_type=jnp.float32) + b_ref[...]
    o_ref[...] = yt
    s = jnp.sum(yt, axis=1, keepdims=True)
    s2 = jnp.sum(yt * yt, axis=1, keepdims=True)
    st = jnp.concatenate([s, s2], axis=1)

    @pl.when(pl.program_id(0) == 0)
    def _():
        st_ref[...] = st

    @pl.when(pl.program_id(0) != 0)
    def _():
        st_ref[...] += st


def _layer_t(yprev, stin, gam, bet, w, bcol, ntot):
    R, Cin = yprev.shape
    Cout = w.shape[0]
    return pl.pallas_call(
        functools.partial(_layer_t_body, ntot),
        grid=(R // BLKR,),
        in_specs=[
            pl.BlockSpec((BLKR, Cin), lambda i: (i, 0)),
            pl.BlockSpec((2, Cin), lambda i: (0, 0)),
            pl.BlockSpec((1, Cin), lambda i: (0, 0)),
            pl.BlockSpec((1, Cin), lambda i: (0, 0)),
            pl.BlockSpec((Cout, Cin), lambda i: (0, 0)),
            pl.BlockSpec((Cout, 1), lambda i: (0, 0)),
        ],
        out_specs=[
            pl.BlockSpec((Cout, BLKR), lambda i: (0, i)),
            pl.BlockSpec((Cout, 2), lambda i: (0, 0)),
        ],
        out_shape=[
            jax.ShapeDtypeStruct((Cout, R), jnp.float32),
            jax.ShapeDtypeStruct((Cout, 2), jnp.float32),
        ],
        compiler_params=pltpu.CompilerParams(
            dimension_semantics=("arbitrary",)),
    )(yprev, stin, gam, bet, w, bcol)


def _final_t_body(ntot, y_ref, stin_ref, gam_ref, bet_ref, o_ref):
    mean = stin_ref[:, 0:1] * (1.0 / ntot)
    ex2 = stin_ref[:, 1:2] * (1.0 / ntot)
    s = gam_ref[...] * lax.rsqrt(ex2 - mean * mean + 1e-3)
    t = bet_ref[...] - mean * s
    o_ref[0] = jnp.maximum(y_ref[...] * s + t, 0.0)


def _final_t(yt, stin, gamcol, betcol, B, N2, ntot):
    C, R = yt.shape
    jb = N2 // BLKR
    return pl.pallas_call(
        functools.partial(_final_t_body, ntot),
        grid=(R // BLKR,),
        in_specs=[
            pl.BlockSpec((C, BLKR), lambda i: (0, i)),
            pl.BlockSpec((C, 2), lambda i: (0, 0)),
            pl.BlockSpec((C, 1), lambda i: (0, 0)),
            pl.BlockSpec((C, 1), lambda i: (0, 0)),
        ],
        out_specs=pl.BlockSpec((1, C, BLKR), lambda i: (i // jb, 0, i % jb)),
        out_shape=jax.ShapeDtypeStruct((B, C, N2), jnp.float32),
        compiler_params=pltpu.CompilerParams(
            dimension_semantics=("parallel",)),
    )(yt, stin, gamcol, betcol)


def kernel(points1, points2, features1, features2, skip_features,
           W1, b1, g1, be1, W2, b2, g2, be2, W3, b3, g3, be3):
    B, _, N1 = points1.shape
    N2 = points2.shape[2]
    C1 = features1.shape[1]
    C2 = features2.shape[1]
    ntot = float(B * N2)

    ptab = _proj(features1, W1[:, :C1].T, 512)
    w2t = W1[:, C1:C1 + C2].T
    w3t = W1[:, C1 + C2:].T
    b1r = b1.reshape(1, -1)

    NS = 8
    Bs = B // NS
    y1a_s = []
    for h in range(NS):
        gidx, w = _topk(points2, points1, Bs * h, Bs)
        wexp = jnp.broadcast_to(w.reshape(-1, 1), (Bs * N2 * K, 16))
        y1a_s.append(_interp_sc(ptab, gidx.reshape(-1), wexp))

    l1 = [_layer1(y1a_s[h], features2, skip_features, w2t, w3t, b1r, Bs * h)
          for h in range(NS)]
    st1 = sum(x[1] for x in l1)
    l2 = [_layer(l1[h][0], st1, g1.reshape(1, -1), be1.reshape(1, -1),
                 W2.T, b2.reshape(1, -1), ntot) for h in range(NS)]
    st2 = sum(x[1] for x in l2)
    l3 = [_layer_t(l2[h][0], st2, g2.reshape(1, -1), be2.reshape(1, -1),
                   W3, b3.reshape(-1, 1), ntot) for h in range(NS)]
    st3 = sum(x[1] for x in l3)
    outs = [_final_t(l3[h][0], st3, g3.reshape(-1, 1), be3.reshape(-1, 1),
                     Bs, N2, ntot) for h in range(NS)]
    return jnp.concatenate(outs, axis=0)


# speedup vs baseline: 1.0383x; 1.0383x over previous
import functools

import jax
import jax.numpy as jnp
from jax import lax
from jax.experimental import pallas as pl
from jax.experimental.pallas import tpu as pltpu
from jax.experimental.pallas import tpu_sc as plsc

K = 3
BLKQ = 512
BLKR = 1024
NW = 32
CQ = 32
NBUF = 2


def _topk_body(n1_total, bs, p2_ref, p1_ref, idx_ref, w_ref):
    b = pl.program_id(0) + bs
    p2b = p2_ref[0]
    p1b = p1_ref[0]
    cross = lax.dot_general(p2b, p1b, (((0,), (0,)), ((), ())),
                            preferred_element_type=jnp.float32)
    n2sq = jnp.sum(p2b * p2b, axis=0)[:, None]
    n1sq = jnp.sum(p1b * p1b, axis=0)[None, :]
    d = jnp.maximum(n2sq + n1sq - 2.0 * cross, 0.0)
    iota = lax.broadcasted_iota(jnp.int32, d.shape, 1)
    idxs, gds = [], []
    for _ in range(K):
        m = jnp.min(d, axis=1, keepdims=True)
        ik = jnp.min(jnp.where(d == m, iota, n1_total), axis=1, keepdims=True)
        idxs.append(ik)
        gds.append(jnp.sqrt(m))
        d = jnp.where(iota == ik, jnp.inf, d)
    inv = [1.0 / (g + 1e-10) for g in gds]
    tot = inv[0] + inv[1] + inv[2]
    idx_ref[0] = jnp.concatenate(idxs, axis=1) + b * n1_total
    w_ref[0] = jnp.concatenate([v / tot for v in inv], axis=1)


def _topk(points2, points1, bs, nb):
    _, _, N2 = points2.shape
    N1 = points1.shape[2]
    grid = (nb, N2 // BLKQ)
    return pl.pallas_call(
        functools.partial(_topk_body, N1, bs),
        grid=grid,
        in_specs=[
            pl.BlockSpec((1, 3, BLKQ), lambda b, j: (b + bs, 0, j)),
            pl.BlockSpec((1, 3, N1), lambda b, j: (b + bs, 0, 0)),
        ],
        out_specs=[
            pl.BlockSpec((1, BLKQ, K), lambda b, j: (b, j, 0)),
            pl.BlockSpec((1, BLKQ, K), lambda b, j: (b, j, 0)),
        ],
        out_shape=[
            jax.ShapeDtypeStruct((nb, N2, K), jnp.int32),
            jax.ShapeDtypeStruct((nb, N2, K), jnp.float32),
        ],
        compiler_params=pltpu.CompilerParams(
            dimension_semantics=("parallel", "parallel")),
    )(points2, points1)


def _proj_body(f1_ref, wt_ref, o_ref):
    o_ref[...] = lax.dot_general(f1_ref[0], wt_ref[...],
                                 (((0,), (0,)), ((), ())),
                                 preferred_element_type=jnp.float32)


def _proj(features1, wt, blk):
    B, C1, N1 = features1.shape
    Cout = wt.shape[1]
    jb = N1 // blk
    return pl.pallas_call(
        _proj_body,
        grid=(B, jb),
        in_specs=[
            pl.BlockSpec((1, C1, blk), lambda b, j: (b, 0, j)),
            pl.BlockSpec((C1, Cout), lambda b, j: (0, 0)),
        ],
        out_specs=pl.BlockSpec((blk, Cout), lambda b, j: (b * jb + j, 0)),
        out_shape=jax.ShapeDtypeStruct((B * N1, Cout), jnp.float32),
        compiler_params=pltpu.CompilerParams(
            dimension_semantics=("parallel", "parallel")),
    )(features1, wt)


def _interp_sc(table, gidx, wexp):
    V, C = table.shape
    QK = gidx.shape[0]
    Q = QK // K
    qw = Q // NW
    nch = qw // CQ
    rows = CQ * K
    cvec = C // 16

    mesh = plsc.VectorSubcoreMesh(core_axis_name="c", subcore_axis_name="s")

    @functools.partial(
        pl.kernel,
        out_type=jax.ShapeDtypeStruct((Q, C), jnp.float32),
        mesh=mesh,
        scratch_types=[
            pltpu.VMEM((qw * K,), jnp.int32),
            [pltpu.VMEM((rows, 16), jnp.float32) for _ in range(NBUF)],
            [pltpu.VMEM((rows, C), jnp.float32) for _ in range(NBUF)],
            [pltpu.SemaphoreType.DMA for _ in range(3 * NBUF)],
        ],
    )
    def k(table_hbm, gidx_hbm, wexp_hbm, out_hbm, idx_all, wbufs, rbufs,
          sems):
        wid = lax.axis_index("s") * 2 + lax.axis_index("c")
        base_q = wid * qw
        gsems = sems[0:NBUF]
        wsems = sems[NBUF:2 * NBUF]
        osems = sems[2 * NBUF:3 * NBUF]
        pltpu.sync_copy(gidx_hbm.at[pl.ds(base_q * K, qw * K)], idx_all)

        def start(ci, sl):
            r0 = ci * rows
            pltpu.async_copy(table_hbm.at[idx_all.at[pl.ds(r0, rows)]],
                             rbufs[sl], gsems[sl])
            pltpu.async_copy(wexp_hbm.at[pl.ds(base_q * K + r0, rows)],
                             wbufs[sl], wsems[sl])

        def wait_in(ci, sl):
            r0 = ci * rows
            pltpu.make_async_copy(table_hbm.at[idx_all.at[pl.ds(r0, rows)]],
                                  rbufs[sl], gsems[sl]).wait()
            pltpu.make_async_copy(wexp_hbm.at[pl.ds(base_q * K + r0, rows)],
                                  wbufs[sl], wsems[sl]).wait()

        def compute(ci, sl):
            rv, wv = rbufs[sl], wbufs[sl]

            def per_q(q, c2):
                rq = q * K
                wa = wv[rq, :]
                wb = wv[rq + 1, :]
                wc = wv[rq + 2, :]
                for c in range(cvec):
                    s = pl.ds(16 * c, 16)
                    rv[q, s] = (wa * rv[rq, s] + wb * rv[rq + 1, s]
                                + wc * rv[rq + 2, s])
                return c2

            lax.fori_loop(0, CQ, per_q, 0)
            pltpu.async_copy(rbufs[sl].at[pl.ds(0, CQ)],
                             out_hbm.at[pl.ds(base_q + ci * CQ, CQ)],
                             osems[sl])

        def wait_out(sl):
            pltpu.make_async_copy(rbufs[sl].at[pl.ds(0, CQ)],
                                  out_hbm.at[pl.ds(base_q, CQ)],
                                  osems[sl]).wait()

        for sl in range(NBUF):
            start(sl, sl)

        def group(ig, c2):
            for sl in range(NBUF):
                ci = NBUF * ig + sl
                wait_in(ci, sl)
                compute(ci, sl)

                @pl.when(ci + NBUF < nch)
                def _():
                    wait_out(sl)
                    start(ci + NBUF, sl)

            return c2

        lax.fori_loop(0, nch // NBUF, group, 0)
        for sl in range(NBUF):
            wait_out(sl)

    return k(table, gidx, wexp)


def _accum_stats(st_ref, y):
    s = jnp.sum(y, axis=0, keepdims=True)
    s2 = jnp.sum(y * y, axis=0, keepdims=True)
    st = jnp.concatenate([s, s2], axis=0)

    @pl.when(pl.program_id(0) == 0)
    def _():
        st_ref[...] = st

    @pl.when(pl.program_id(0) != 0)
    def _():
        st_ref[...] += st


def _l1_body(jb, ya_ref, f2_ref, sk_ref, w2t_ref, w3t_ref, b_ref, o_ref,
             st_ref):
    dn = (((0,), (0,)), ((), ()))
    y = (ya_ref[...]
         + lax.dot_general(f2_ref[0], w2t_ref[...], dn,
                           preferred_element_type=jnp.float32)
         + lax.dot_general(sk_ref[0], w3t_ref[...], dn,
                           preferred_element_type=jnp.float32)
         + b_ref[...])
    o_ref[...] = y
    _accum_stats(st_ref, y)


def _layer1(ya, features2, skip_features, w2t, w3t, b, bs):
    R, C = ya.shape
    B, C2, N2 = features2.shape
    CS = skip_features.shape[1]
    jb = N2 // BLKR
    return pl.pallas_call(
        functools.partial(_l1_body, jb),
        grid=(R // BLKR,),
        in_specs=[
            pl.BlockSpec((BLKR, C), lambda i: (i, 0)),
            pl.BlockSpec((1, C2, BLKR), lambda i: (i // jb + bs, 0, i % jb)),
            pl.BlockSpec((1, CS, BLKR), lambda i: (i // jb + bs, 0, i % jb)),
            pl.BlockSpec((C2, C), lambda i: (0, 0)),
            pl.BlockSpec((CS, C), lambda i: (0, 0)),
            pl.BlockSpec((1, C), lambda i: (0, 0)),
        ],
        out_specs=[
            pl.BlockSpec((BLKR, C), lambda i: (i, 0)),
            pl.BlockSpec((2, C), lambda i: (0, 0)),
        ],
        out_shape=[
            jax.ShapeDtypeStruct((R, C), jnp.float32),
            jax.ShapeDtypeStruct((2, C), jnp.float32),
        ],
        compiler_params=pltpu.CompilerParams(
            dimension_semantics=("arbitrary",)),
    )(ya, features2, skip_features, w2t, w3t, b)


def _bn_relu(y_ref, stin_ref, gam_ref, bet_ref, ntot):
    mean = stin_ref[0:1, :] * (1.0 / ntot)
    ex2 = stin_ref[1:2, :] * (1.0 / ntot)
    s = gam_ref[...] * lax.rsqrt(ex2 - mean * mean + 1e-3)
    t = bet_ref[...] - mean * s
    return jnp.maximum(y_ref[...] * s + t, 0.0)


def _layer_body(ntot, y_ref, stin_ref, gam_ref, bet_ref, wt_ref, b_ref,
                o_ref, st_ref):
    x = _bn_relu(y_ref, stin_ref, gam_ref, bet_ref, ntot)
    y = jnp.dot(x, wt_ref[...], preferred_element_type=jnp.float32) + b_ref[...]
    o_ref[...] = y
    _accum_stats(st_ref, y)


def _layer(yprev, stin, gam, bet, wt, b, ntot):
    R, Cin = yprev.shape
    Cout = wt.shape[1]
    return pl.pallas_call(
        functools.partial(_layer_body, ntot),
        grid=(R // BLKR,),
        in_specs=[
            pl.BlockSpec((BLKR, Cin), lambda i: (i, 0)),
            pl.BlockSpec((2, Cin), lambda i: (0, 0)),
            pl.BlockSpec((1, Cin), lambda i: (0, 0)),
            pl.BlockSpec((1, Cin), lambda i: (0, 0)),
            pl.BlockSpec((Cin, Cout), lambda i: (0, 0)),
            pl.BlockSpec((1, Cout), lambda i: (0, 0)),
        ],
        out_specs=[
            pl.BlockSpec((BLKR, Cout), lambda i: (i, 0)),
            pl.BlockSpec((2, Cout), lambda i: (0, 0)),
        ],
        out_shape=[
            jax.ShapeDtypeStruct((R, Cout), jnp.float32),
            jax.ShapeDtypeStruct((2, Cout), jnp.float32),
        ],
        compiler_params=pltpu.CompilerParams(
            dimension_semantics=("arbitrary",)),
    )(yprev, stin, gam, bet, wt, b)


def _layer_t_body(ntot, y_ref, stin_ref, gam_ref, bet_ref, w_ref, b_ref,
                  o_ref, st_ref):
    x = _bn_relu(y_ref, stin_ref, gam_ref, bet_ref, ntot)
    yt = lax.dot_general(w_ref[...], x, (((1,), (1,)), ((), ())),
                         preferred_element_type=jnp.float32) + b_ref[...]
    o_ref[...] = yt
    s = jnp.sum(yt, axis=1, keepdims=True)
    s2 = jnp.sum(yt * yt, axis=1, keepdims=True)
    st = jnp.concatenate([s, s2], axis=1)

    @pl.when(pl.program_id(0) == 0)
    def _():
        st_ref[...] = st

    @pl.when(pl.program_id(0) != 0)
    def _():
        st_ref[...] += st


def _layer_t(yprev, stin, gam, bet, w, bcol, ntot):
    R, Cin = yprev.shape
    Cout = w.shape[0]
    return pl.pallas_call(
        functools.partial(_layer_t_body, ntot),
        grid=(R // BLKR,),
        in_specs=[
            pl.BlockSpec((BLKR, Cin), lambda i: (i, 0)),
            pl.BlockSpec((2, Cin), lambda i: (0, 0)),
            pl.BlockSpec((1, Cin), lambda i: (0, 0)),
            pl.BlockSpec((1, Cin), lambda i: (0, 0)),
            pl.BlockSpec((Cout, Cin), lambda i: (0, 0)),
            pl.BlockSpec((Cout, 1), lambda i: (0, 0)),
        ],
        out_specs=[
            pl.BlockSpec((Cout, BLKR), lambda i: (0, i)),
            pl.BlockSpec((Cout, 2), lambda i: (0, 0)),
        ],
        out_shape=[
            jax.ShapeDtypeStruct((Cout, R), jnp.float32),
            jax.ShapeDtypeStruct((Cout, 2), jnp.float32),
        ],
        compiler_params=pltpu.CompilerParams(
            dimension_semantics=("arbitrary",)),
    )(yprev, stin, gam, bet, w, bcol)


def _final_t_body(ntot, y_ref, stin_ref, gam_ref, bet_ref, o_ref):
    mean = stin_ref[:, 0:1] * (1.0 / ntot)
    ex2 = stin_ref[:, 1:2] * (1.0 / ntot)
    s = gam_ref[...] * lax.rsqrt(ex2 - mean * mean + 1e-3)
    t = bet_ref[...] - mean * s
    o_ref[0] = jnp.maximum(y_ref[...] * s + t, 0.0)


def _final_t(yt, stin, gamcol, betcol, B, N2, ntot):
    C, R = yt.shape
    jb = N2 // BLKR
    return pl.pallas_call(
        functools.partial(_final_t_body, ntot),
        grid=(R // BLKR,),
        in_specs=[
            pl.BlockSpec((C, BLKR), lambda i: (0, i)),
            pl.BlockSpec((C, 2), lambda i: (0, 0)),
            pl.BlockSpec((C, 1), lambda i: (0, 0)),
            pl.BlockSpec((C, 1), lambda i: (0, 0)),
        ],
        out_specs=pl.BlockSpec((1, C, BLKR), lambda i: (i // jb, 0, i % jb)),
        out_shape=jax.ShapeDtypeStruct((B, C, N2), jnp.float32),
        compiler_params=pltpu.CompilerParams(
            dimension_semantics=("parallel",)),
    )(yt, stin, gamcol, betcol)


def kernel(points1, points2, features1, features2, skip_features,
           W1, b1, g1, be1, W2, b2, g2, be2, W3, b3, g3, be3):
    B, _, N1 = points1.shape
    N2 = points2.shape[2]
    C1 = features1.shape[1]
    C2 = features2.shape[1]
    ntot = float(B * N2)

    ptab = _proj(features1, W1[:, :C1].T, 512)
    w2t = W1[:, C1:C1 + C2].T
    w3t = W1[:, C1 + C2:].T
    b1r = b1.reshape(1, -1)

    NS = 4
    Bs = B // NS
    y1a_s = []
    for h in range(NS):
        gidx, w = _topk(points2, points1, Bs * h, Bs)
        wexp = jnp.broadcast_to(w.reshape(-1, 1), (Bs * N2 * K, 16))
        y1a_s.append(_interp_sc(ptab, gidx.reshape(-1), wexp))

    l1 = [_layer1(y1a_s[h], features2, skip_features, w2t, w3t, b1r, Bs * h)
          for h in range(NS)]
    st1 = sum(x[1] for x in l1)
    l2 = [_layer(l1[h][0], st1, g1.reshape(1, -1), be1.reshape(1, -1),
                 W2.T, b2.reshape(1, -1), ntot) for h in range(NS)]
    st2 = sum(x[1] for x in l2)
    l3 = [_layer_t(l2[h][0], st2, g2.reshape(1, -1), be2.reshape(1, -1),
                   W3, b3.reshape(-1, 1), ntot) for h in range(NS)]
    st3 = sum(x[1] for x in l3)
    outs = [_final_t(l3[h][0], st3, g3.reshape(-1, 1), be3.reshape(-1, 1),
                     Bs, N2, ntot) for h in range(NS)]
    return jnp.concatenate(outs, axis=0)
